# baseline (speedup 1.0000x reference)
import functools

import jax
import jax.numpy as jnp
from jax import lax
from jax.experimental import pallas as pl
from jax.experimental.pallas import tpu as pltpu
from jax.experimental.pallas import tpu_sc as plsc

VOCAB = 1000000
EMBED = 64
BATCH = 16384
NEG = 10

NUM_CORES = 2
NUM_SUBCORES = 16
NUM_WORKERS = NUM_CORES * NUM_SUBCORES
ITEMS_PER_WORKER = BATCH // NUM_WORKERS
SUB = 64
NSUB = ITEMS_PER_WORKER // SUB
NCHUNK = EMBED // 16

NBLK = VOCAB // 128
TAIL = VOCAB - NBLK * 128
BLK_PER_W = (NBLK + NUM_WORKERS - 1) // NUM_WORKERS

POS_PART = BATCH * 16
TOTAL_PART = (BATCH + BATCH * NEG) * 16
PART_ROWS = TOTAL_PART // 128


def _sc_transpose_kernel(src_hbm, tail_hbm, out_hbm, blk, t2):
    wid = lax.axis_index("s") * NUM_CORES + lax.axis_index("c")
    lane = lax.iota(jnp.int32, 16)

    def do_block(b):
        v0 = b * 128
        pltpu.sync_copy(src_hbm.at[:, pl.ds(v0, 128)], blk)

        def d_body(d, _):
            for g in range(8):
                v = g * 16 + lane
                vec = blk[d, pl.ds(g * 16, 16)]
                plsc.store_scatter(t2, [v >> 1, (v & 1) * 64 + d], vec)
            return 0

        lax.fori_loop(0, EMBED, d_body, 0)
        pltpu.sync_copy(t2, out_hbm.at[pl.ds(b * 64, 64), :])

    def blk_body(t, _):
        b = wid + t * NUM_WORKERS

        @pl.when(b < NBLK)
        def _():
            do_block(b)
        return 0

    lax.fori_loop(0, BLK_PER_W, blk_body, 0)

    @pl.when(wid == NUM_WORKERS - 1)
    def _():
        pltpu.sync_copy(tail_hbm, t2.at[pl.ds(0, TAIL // 2), :])
        pltpu.sync_copy(t2.at[pl.ds(0, TAIL // 2), :],
                        out_hbm.at[pl.ds(NBLK * 64, TAIL // 2), :])


def _sc_scores_kernel(word_hbm, cidx_hbm, clsb_hbm, nidx_hbm, nlsb_hbm,
                      wtab_hbm, ctab_hbm,
                      part_out,
                      widx, cidx, clsb, nidx, nlsb,
                      xrows, yrows, nrows, ppart, npart, sem, semr):
    wid = lax.axis_index("s") * NUM_CORES + lax.axis_index("c")
    base_w = wid * ITEMS_PER_WORKER

    pltpu.sync_copy(word_hbm.at[pl.ds(base_w, ITEMS_PER_WORKER)],
                    widx.at[pl.ds(0, ITEMS_PER_WORKER)])
    pltpu.sync_copy(cidx_hbm.at[pl.ds(base_w, ITEMS_PER_WORKER)], cidx)
    pltpu.sync_copy(clsb_hbm.at[pl.ds(base_w, ITEMS_PER_WORKER)],
                    clsb.at[pl.ds(0, ITEMS_PER_WORKER)])
    for k in range(NEG):
        pltpu.sync_copy(nidx_hbm.at[pl.ds(k * BATCH + base_w, ITEMS_PER_WORKER)],
                        nidx.at[k])
        pltpu.sync_copy(nlsb_hbm.at[pl.ds(k * BATCH + base_w, ITEMS_PER_WORKER)],
                        nlsb.at[k, pl.ds(0, ITEMS_PER_WORKER)])

    for c in range(NSUB):
        lo = c * SUB

        cps = [pltpu.async_copy(ctab_hbm.at[cidx.at[pl.ds(lo, SUB)]], yrows, sem)]
        for k in range(NEG):
            cps.append(pltpu.async_copy(
                ctab_hbm.at[nidx.at[k, pl.ds(lo, SUB)]], nrows.at[k], sem))

        def enq_body(i, _):
            rw = widx[pl.ds(lo + i, 16)][0]
            pltpu.async_copy(wtab_hbm.at[rw], xrows.at[i], semr)
            return 0

        lax.fori_loop(0, SUB, enq_body, 0)
        pltpu.make_async_copy(wtab_hbm.at[pl.ds(0, SUB), :], xrows, semr).wait()
        for cp in cps:
            cp.wait()

        def item_body(i, _):
            g = lo + i
            xs = [xrows[i, pl.ds(j * 16, 16)] for j in range(NCHUNK)]
            hy = clsb[pl.ds(g, 16)][0] * 64
            acc = xs[0] * yrows[i, pl.ds(hy, 16)]
            for j in range(1, NCHUNK):
                acc = acc + xs[j] * yrows[i, pl.ds(hy + j * 16, 16)]
            ppart[pl.ds(i * 16, 16)] = acc
            for k in range(NEG):
                hn = nlsb[k, pl.ds(g, 16)][0] * 64
                acc = xs[0] * nrows[k, i, pl.ds(hn, 16)]
                for j in range(1, NCHUNK):
                    acc = acc + xs[j] * nrows[k, i, pl.ds(hn + j * 16, 16)]

                npart[pl.ds((i * NEG + k) * 16, 16)] = acc
            return 0

        lax.fori_loop(0, SUB, item_body, 0)

        base_c = base_w + lo
        pltpu.sync_copy(ppart, part_out.at[pl.ds(base_c * 16, SUB * 16)])
        pltpu.sync_copy(
            npart,
            part_out.at[pl.ds(POS_PART + base_c * NEG * 16, SUB * NEG * 16)])


def _loss_body(part_ref, out_ref):
    x = part_ref[...]
    r = lax.broadcasted_iota(jnp.int32, (128, 8), 0) // 16
    g = lax.broadcasted_iota(jnp.int32, (128, 8), 1)
    gmat = (r == g).astype(jnp.float32)
    s = jax.lax.dot_general(x, gmat, (((1,), (0,)), ((), ())),
                            preferred_element_type=jnp.float32)
    row = lax.broadcasted_iota(jnp.int32, (PART_ROWS, 8), 0)
    z = jnp.where(row < (BATCH * 16) // 128, s, -s)
    l = jnp.minimum(z, 0.0) - jnp.log1p(jnp.exp(-jnp.abs(z)))
    out_ref[...] = jnp.full((1, 1), -jnp.sum(l) / BATCH, jnp.float32)


def kernel(word, context, negative_samples, word_embed, ctx_embed):
    negt = negative_samples.T.reshape(-1)

    mesh = plsc.VectorSubcoreMesh(core_axis_name="c", subcore_axis_name="s")
    params = pltpu.CompilerParams(
        needs_layout_passes=False, use_tc_tiling_on_sc=True)

    tr = functools.partial(
        pl.kernel,
        mesh=mesh,
        compiler_params=params,
        out_type=jax.ShapeDtypeStruct((VOCAB // 2, 128), jnp.float32),
        scratch_types=[
            pltpu.VMEM((EMBED, 128), jnp.float32),
            pltpu.VMEM((EMBED, 128), jnp.float32),
        ],
    )(_sc_transpose_kernel)
    tail = lax.slice(ctx_embed, (NBLK * 128, 0), (VOCAB, EMBED))
    ctab = tr(ctx_embed.T, tail.reshape(TAIL // 2, 128))

    sc = functools.partial(
        pl.kernel,
        mesh=mesh,
        compiler_params=params,
        out_type=jax.ShapeDtypeStruct((TOTAL_PART,), jnp.float32),
        scratch_types=[
            pltpu.VMEM((ITEMS_PER_WORKER + 16,), jnp.int32),
            pltpu.VMEM((ITEMS_PER_WORKER,), jnp.int32),
            pltpu.VMEM((ITEMS_PER_WORKER + 16,), jnp.int32),
            pltpu.VMEM((NEG, ITEMS_PER_WORKER), jnp.int32),
            pltpu.VMEM((NEG, ITEMS_PER_WORKER + 16), jnp.int32),
            pltpu.VMEM((SUB, EMBED), jnp.float32),
            pltpu.VMEM((SUB, 2 * EMBED), jnp.float32),
            pltpu.VMEM((NEG, SUB, 2 * EMBED), jnp.float32),
            pltpu.VMEM((SUB * 16,), jnp.float32),
            pltpu.VMEM((SUB * NEG * 16,), jnp.float32),
            pltpu.SemaphoreType.DMA,
            pltpu.SemaphoreType.DMA,
        ],
    )(_sc_scores_kernel)
    part = sc(word, context >> 1, context & 1, negt >> 1, negt & 1,
              word_embed, ctab)

    loss2d = pl.pallas_call(
        _loss_body,
        out_shape=jax.ShapeDtypeStruct((1, 1), jnp.float32),
    )(part.reshape(PART_ROWS, 128))
    return loss2d[0, 0]


# speedup vs baseline: 1.2536x; 1.2536x over previous
import functools

import jax
import jax.numpy as jnp
from jax import lax
from jax.experimental import pallas as pl
from jax.experimental.pallas import tpu as pltpu
from jax.experimental.pallas import tpu_sc as plsc

VOCAB = 1000000
EMBED = 64
BATCH = 16384
NEG = 10

NUM_CORES = 2
NUM_SUBCORES = 16
NUM_WORKERS = NUM_CORES * NUM_SUBCORES
ITEMS_PER_WORKER = BATCH // NUM_WORKERS
SUB = 64
NSUB = ITEMS_PER_WORKER // SUB
NCHUNK = EMBED // 16

SBV = 256
NSB = (VOCAB // 128) * 128 // SBV
TAIL = VOCAB - NSB * SBV
SB_PER_W = (NSB + NUM_WORKERS - 1) // NUM_WORKERS
SB_PAIRS = (SB_PER_W + 1) // 2

POS_PART = BATCH * 16
TOTAL_PART = (BATCH + BATCH * NEG) * 16
PART_ROWS = TOTAL_PART // 128


def _sc_transpose_kernel(src_hbm, tail_hbm, out_hbm,
                         blk_a, blk_b, t2_a, t2_b,
                         sem_a, sem_b, semo_a, semo_b):
    wid = lax.axis_index("s") * NUM_CORES + lax.axis_index("c")
    lane = lax.iota(jnp.int32, 16)

    def start_in(t, blk, sem):
        b = wid + t * NUM_WORKERS

        @pl.when(b < NSB)
        def _():
            pltpu.async_copy(src_hbm.at[:, pl.ds(b * SBV, SBV)], blk, sem)

    def phase(t, blk, sem, nblk, nsem, t2, semo):
        b = wid + t * NUM_WORKERS

        @pl.when(b < NSB)
        def _():
            start_in(t + 1, nblk, nsem)
            pltpu.make_async_copy(src_hbm.at[:, pl.ds(0, SBV)], blk, sem).wait()

            @pl.when(t >= 2)
            def _():
                pltpu.make_async_copy(
                    t2, out_hbm.at[pl.ds(0, SBV // 2), :], semo).wait()

            def d_body(d, _):
                for s in range(SBV // 128):
                    for g in range(8):
                        v = s * 128 + g * 16 + lane
                        vec = blk[d, pl.ds(s * 128 + g * 16, 16)]
                        plsc.store_scatter(t2, [v >> 1, (v & 1) * 64 + d], vec)
                return 0

            lax.fori_loop(0, EMBED, d_body, 0)
            pltpu.async_copy(t2, out_hbm.at[pl.ds(b * (SBV // 2), SBV // 2), :],
                             semo)

    start_in(0, blk_a, sem_a)

    def pair_body(tt, _):
        phase(tt * 2, blk_a, sem_a, blk_b, sem_b, t2_a, semo_a)
        phase(tt * 2 + 1, blk_b, sem_b, blk_a, sem_a, t2_b, semo_b)
        return 0

    lax.fori_loop(0, SB_PAIRS, pair_body, 0)
    pltpu.make_async_copy(t2_a, out_hbm.at[pl.ds(0, SBV // 2), :], semo_a).wait()
    pltpu.make_async_copy(t2_b, out_hbm.at[pl.ds(0, SBV // 2), :], semo_b).wait()

    @pl.when(wid == NUM_WORKERS - 1)
    def _():
        pltpu.sync_copy(tail_hbm, t2_a.at[pl.ds(0, TAIL // 2), :])
        pltpu.sync_copy(t2_a.at[pl.ds(0, TAIL // 2), :],
                        out_hbm.at[pl.ds(NSB * (SBV // 2), TAIL // 2), :])


def _sc_scores_kernel(word_hbm, cidx_hbm, clsb_hbm, nidx_hbm, nlsb_hbm,
                      wtab_hbm, ctab_hbm,
                      part_out,
                      widx, cidx, clsb, nidx, nlsb,
                      xrows, yrows, nrows, ppart, npart, sem, semr):
    wid = lax.axis_index("s") * NUM_CORES + lax.axis_index("c")
    base_w = wid * ITEMS_PER_WORKER

    pltpu.sync_copy(word_hbm.at[pl.ds(base_w, ITEMS_PER_WORKER)],
                    widx.at[pl.ds(0, ITEMS_PER_WORKER)])
    pltpu.sync_copy(cidx_hbm.at[pl.ds(base_w, ITEMS_PER_WORKER)], cidx)
    pltpu.sync_copy(clsb_hbm.at[pl.ds(base_w, ITEMS_PER_WORKER)],
                    clsb.at[pl.ds(0, ITEMS_PER_WORKER)])
    for k in range(NEG):
        pltpu.sync_copy(nidx_hbm.at[pl.ds(k * BATCH + base_w, ITEMS_PER_WORKER)],
                        nidx.at[k])
        pltpu.sync_copy(nlsb_hbm.at[pl.ds(k * BATCH + base_w, ITEMS_PER_WORKER)],
                        nlsb.at[k, pl.ds(0, ITEMS_PER_WORKER)])

    for c in range(NSUB):
        lo = c * SUB

        cps = [pltpu.async_copy(ctab_hbm.at[cidx.at[pl.ds(lo, SUB)]], yrows, sem)]
        for k in range(NEG):
            cps.append(pltpu.async_copy(
                ctab_hbm.at[nidx.at[k, pl.ds(lo, SUB)]], nrows.at[k], sem))

        def enq_body(i, _):
            rw = widx[pl.ds(lo + i, 16)][0]
            pltpu.async_copy(wtab_hbm.at[rw], xrows.at[i], semr)
            return 0

        lax.fori_loop(0, SUB, enq_body, 0)
        pltpu.make_async_copy(wtab_hbm.at[pl.ds(0, SUB), :], xrows, semr).wait()
        for cp in cps:
            cp.wait()

        def item_body(i, _):
            g = lo + i
            xs = [xrows[i, pl.ds(j * 16, 16)] for j in range(NCHUNK)]
            hy = clsb[pl.ds(g, 16)][0] * 64
            acc = xs[0] * yrows[i, pl.ds(hy, 16)]
            for j in range(1, NCHUNK):
                acc = acc + xs[j] * yrows[i, pl.ds(hy + j * 16, 16)]
            ppart[pl.ds(i * 16, 16)] = acc
            for k in range(NEG):
                hn = nlsb[k, pl.ds(g, 16)][0] * 64
                acc = xs[0] * nrows[k, i, pl.ds(hn, 16)]
                for j in range(1, NCHUNK):
                    acc = acc + xs[j] * nrows[k, i, pl.ds(hn + j * 16, 16)]

                npart[pl.ds((i * NEG + k) * 16, 16)] = acc
            return 0

        lax.fori_loop(0, SUB, item_body, 0)

        base_c = base_w + lo
        pltpu.sync_copy(ppart, part_out.at[pl.ds(base_c * 16, SUB * 16)])
        pltpu.sync_copy(
            npart,
            part_out.at[pl.ds(POS_PART + base_c * NEG * 16, SUB * NEG * 16)])


def _loss_body(part_ref, out_ref):
    x = part_ref[...]
    r = lax.broadcasted_iota(jnp.int32, (128, 8), 0) // 16
    g = lax.broadcasted_iota(jnp.int32, (128, 8), 1)
    gmat = (r == g).astype(jnp.float32)
    s = jax.lax.dot_general(x, gmat, (((1,), (0,)), ((), ())),
                            preferred_element_type=jnp.float32)
    row = lax.broadcasted_iota(jnp.int32, (PART_ROWS, 8), 0)
    z = jnp.where(row < (BATCH * 16) // 128, s, -s)
    l = jnp.minimum(z, 0.0) - jnp.log1p(jnp.exp(-jnp.abs(z)))
    out_ref[...] = jnp.full((1, 1), -jnp.sum(l) / BATCH, jnp.float32)


def kernel(word, context, negative_samples, word_embed, ctx_embed):
    negt = negative_samples.T.reshape(-1)

    mesh = plsc.VectorSubcoreMesh(core_axis_name="c", subcore_axis_name="s")
    params = pltpu.CompilerParams(
        needs_layout_passes=False, use_tc_tiling_on_sc=True)

    tr = functools.partial(
        pl.kernel,
        mesh=mesh,
        compiler_params=params,
        out_type=jax.ShapeDtypeStruct((VOCAB // 2, 128), jnp.float32),
        scratch_types=[
            pltpu.VMEM((EMBED, SBV), jnp.float32),
            pltpu.VMEM((EMBED, SBV), jnp.float32),
            pltpu.VMEM((SBV // 2, 128), jnp.float32),
            pltpu.VMEM((SBV // 2, 128), jnp.float32),
            pltpu.SemaphoreType.DMA,
            pltpu.SemaphoreType.DMA,
            pltpu.SemaphoreType.DMA,
            pltpu.SemaphoreType.DMA,
        ],
    )(_sc_transpose_kernel)
    tail = lax.slice(ctx_embed, (NSB * SBV, 0), (VOCAB, EMBED))
    ctab = tr(ctx_embed.T, tail.reshape(TAIL // 2, 128))

    sc = functools.partial(
        pl.kernel,
        mesh=mesh,
        compiler_params=params,
        out_type=jax.ShapeDtypeStruct((TOTAL_PART,), jnp.float32),
        scratch_types=[
            pltpu.VMEM((ITEMS_PER_WORKER + 16,), jnp.int32),
            pltpu.VMEM((ITEMS_PER_WORKER,), jnp.int32),
            pltpu.VMEM((ITEMS_PER_WORKER + 16,), jnp.int32),
            pltpu.VMEM((NEG, ITEMS_PER_WORKER), jnp.int32),
            pltpu.VMEM((NEG, ITEMS_PER_WORKER + 16), jnp.int32),
            pltpu.VMEM((SUB, EMBED), jnp.float32),
            pltpu.VMEM((SUB, 2 * EMBED), jnp.float32),
            pltpu.VMEM((NEG, SUB, 2 * EMBED), jnp.float32),
            pltpu.VMEM((SUB * 16,), jnp.float32),
            pltpu.VMEM((SUB * NEG * 16,), jnp.float32),
            pltpu.SemaphoreType.DMA,
            pltpu.SemaphoreType.DMA,
        ],
    )(_sc_scores_kernel)
    part = sc(word, context >> 1, context & 1, negt >> 1, negt & 1,
              word_embed, ctab)

    loss2d = pl.pallas_call(
        _loss_body,
        out_shape=jax.ShapeDtypeStruct((1, 1), jnp.float32),
    )(part.reshape(PART_ROWS, 128))
    return loss2d[0, 0]


# speedup vs baseline: 2.1071x; 1.6809x over previous
import functools

import jax
import jax.numpy as jnp
from jax import lax
from jax.experimental import pallas as pl
from jax.experimental.pallas import tpu as pltpu
from jax.experimental.pallas import tpu_sc as plsc

VOCAB = 1000000
EMBED = 64
BATCH = 16384
NEG = 10

NUM_CORES = 2
NUM_SUBCORES = 16
NUM_WORKERS = NUM_CORES * NUM_SUBCORES
ITEMS_PER_WORKER = BATCH // NUM_WORKERS
SUB = 32
NSUB = ITEMS_PER_WORKER // SUB
NPAIR = NSUB // 2
NCHUNK = EMBED // 16

POS_PART = BATCH * 16
TOTAL_PART = (BATCH + BATCH * NEG) * 16
PART_ROWS = TOTAL_PART // 128


def _sc_scores_kernel(word_hbm, ctx_hbm, negs_hbm, wtab_hbm, ctab_hbm,
                      part_out,
                      widx, cidx, nidx,
                      xrows_a, yrows_a, nrows_a, xrows_b, yrows_b, nrows_b,
                      ppart, npart, sem_a, sem_b):
    wid = lax.axis_index("s") * NUM_CORES + lax.axis_index("c")
    base_w = wid * ITEMS_PER_WORKER

    pltpu.sync_copy(word_hbm.at[pl.ds(base_w, ITEMS_PER_WORKER)],
                    widx.at[pl.ds(0, ITEMS_PER_WORKER)])
    pltpu.sync_copy(ctx_hbm.at[pl.ds(base_w, ITEMS_PER_WORKER)],
                    cidx.at[pl.ds(0, ITEMS_PER_WORKER)])
    pltpu.sync_copy(negs_hbm.at[pl.ds(base_w * NEG, ITEMS_PER_WORKER * NEG)],
                    nidx.at[pl.ds(0, ITEMS_PER_WORKER * NEG)])

    def enqueue(c, xrows, yrows, nrows, sem):
        lo = c * SUB

        def enq_body(i, _):
            g = lo + i
            rw = widx[pl.ds(g, 16)][0]
            pltpu.async_copy(wtab_hbm.at[rw], xrows.at[i], sem)
            rc = cidx[pl.ds(g, 16)][0]
            pltpu.async_copy(ctab_hbm.at[rc], yrows.at[i], sem)
            for k in range(NEG):
                rn = nidx[pl.ds(g * NEG + k, 16)][0]
                pltpu.async_copy(ctab_hbm.at[rn], nrows.at[i * NEG + k], sem)
            return 0

        lax.fori_loop(0, SUB, enq_body, 0)

    def drain(xrows, yrows, nrows, sem):
        pltpu.make_async_copy(wtab_hbm.at[pl.ds(0, SUB), :], xrows, sem).wait()
        pltpu.make_async_copy(ctab_hbm.at[pl.ds(0, SUB), :], yrows, sem).wait()
        pltpu.make_async_copy(ctab_hbm.at[pl.ds(0, SUB * NEG), :], nrows,
                              sem).wait()

    def compute(c, xrows, yrows, nrows):
        def item_body(i, _):
            xs = [xrows[i, pl.ds(j * 16, 16)] for j in range(NCHUNK)]
            acc = xs[0] * yrows[i, pl.ds(0, 16)]
            for j in range(1, NCHUNK):
                acc = acc + xs[j] * yrows[i, pl.ds(j * 16, 16)]
            ppart[pl.ds(i * 16, 16)] = acc
            for k in range(NEG):
                acc = xs[0] * nrows[i * NEG + k, pl.ds(0, 16)]
                for j in range(1, NCHUNK):
                    acc = acc + xs[j] * nrows[i * NEG + k, pl.ds(j * 16, 16)]

                npart[pl.ds((i * NEG + k) * 16, 16)] = acc
            return 0

        lax.fori_loop(0, SUB, item_body, 0)

        base_c = base_w + c * SUB
        pltpu.sync_copy(ppart, part_out.at[pl.ds(base_c * 16, SUB * 16)])
        pltpu.sync_copy(
            npart,
            part_out.at[pl.ds(POS_PART + base_c * NEG * 16, SUB * NEG * 16)])

    enqueue(0, xrows_a, yrows_a, nrows_a, sem_a)

    def pair_body(tt, _):
        c = tt * 2
        enqueue(c + 1, xrows_b, yrows_b, nrows_b, sem_b)
        drain(xrows_a, yrows_a, nrows_a, sem_a)
        compute(c, xrows_a, yrows_a, nrows_a)

        @pl.when(tt < NPAIR - 1)
        def _():
            enqueue(c + 2, xrows_a, yrows_a, nrows_a, sem_a)
        drain(xrows_b, yrows_b, nrows_b, sem_b)
        compute(c + 1, xrows_b, yrows_b, nrows_b)
        return 0

    lax.fori_loop(0, NPAIR, pair_body, 0)


def _loss_body(part_ref, out_ref):
    x = part_ref[...]
    r = lax.broadcasted_iota(jnp.int32, (128, 8), 0) // 16
    g = lax.broadcasted_iota(jnp.int32, (128, 8), 1)
    gmat = (r == g).astype(jnp.float32)
    s = jax.lax.dot_general(x, gmat, (((1,), (0,)), ((), ())),
                            preferred_element_type=jnp.float32)
    row = lax.broadcasted_iota(jnp.int32, (PART_ROWS, 8), 0)
    z = jnp.where(row < (BATCH * 16) // 128, s, -s)
    l = jnp.minimum(z, 0.0) - jnp.log1p(jnp.exp(-jnp.abs(z)))
    out_ref[...] = jnp.full((1, 1), -jnp.sum(l) / BATCH, jnp.float32)


def kernel(word, context, negative_samples, word_embed, ctx_embed):
    negs = negative_samples.reshape(-1)

    mesh = plsc.VectorSubcoreMesh(core_axis_name="c", subcore_axis_name="s")
    sc = functools.partial(
        pl.kernel,
        mesh=mesh,
        compiler_params=pltpu.CompilerParams(use_tc_tiling_on_sc=True),
        out_type=jax.ShapeDtypeStruct((TOTAL_PART,), jnp.float32),
        scratch_types=[
            pltpu.VMEM((ITEMS_PER_WORKER + 16,), jnp.int32),
            pltpu.VMEM((ITEMS_PER_WORKER + 16,), jnp.int32),
            pltpu.VMEM((ITEMS_PER_WORKER * NEG + 16,), jnp.int32),
            pltpu.VMEM((SUB, EMBED), jnp.float32),
            pltpu.VMEM((SUB, EMBED), jnp.float32),
            pltpu.VMEM((SUB * NEG, EMBED), jnp.float32),
            pltpu.VMEM((SUB, EMBED), jnp.float32),
            pltpu.VMEM((SUB, EMBED), jnp.float32),
            pltpu.VMEM((SUB * NEG, EMBED), jnp.float32),
            pltpu.VMEM((SUB * 16,), jnp.float32),
            pltpu.VMEM((SUB * NEG * 16,), jnp.float32),
            pltpu.SemaphoreType.DMA,
            pltpu.SemaphoreType.DMA,
        ],
    )(_sc_scores_kernel)
    part = sc(word, context, negs, word_embed, ctx_embed)

    loss2d = pl.pallas_call(
        _loss_body,
        out_shape=jax.ShapeDtypeStruct((1, 1), jnp.float32),
    )(part.reshape(PART_ROWS, 128))
    return loss2d[0, 0]


# speedup vs baseline: 2.1211x; 1.0066x over previous
import functools

import jax
import jax.numpy as jnp
from jax import lax
from jax.experimental import pallas as pl
from jax.experimental.pallas import tpu as pltpu
from jax.experimental.pallas import tpu_sc as plsc

VOCAB = 1000000
EMBED = 64
BATCH = 16384
NEG = 10

NUM_CORES = 2
NUM_SUBCORES = 16
NUM_WORKERS = NUM_CORES * NUM_SUBCORES
ITEMS_PER_WORKER = BATCH // NUM_WORKERS
SUB = 64
NSUB = ITEMS_PER_WORKER // SUB
NCHUNK = EMBED // 16

STASH_ROWS = BATCH * (1 + NEG)

POS_PART = BATCH * 16
TOTAL_PART = (BATCH + BATCH * NEG) * 16
PART_ROWS = TOTAL_PART // 128


def _sc_ctx_kernel(ctx_hbm, negs_hbm, ctab_hbm, stash_out,
                   cidx, nidx, yrows, nrows, sem):
    wid = lax.axis_index("s") * NUM_CORES + lax.axis_index("c")
    base_w = wid * ITEMS_PER_WORKER

    pltpu.sync_copy(ctx_hbm.at[pl.ds(base_w, ITEMS_PER_WORKER)],
                    cidx.at[pl.ds(0, ITEMS_PER_WORKER)])
    pltpu.sync_copy(negs_hbm.at[pl.ds(base_w * NEG, ITEMS_PER_WORKER * NEG)],
                    nidx.at[pl.ds(0, ITEMS_PER_WORKER * NEG)])

    for c in range(NSUB):
        lo = c * SUB

        def enq_body(i, _):
            g = lo + i
            rc = cidx[pl.ds(g, 16)][0]
            pltpu.async_copy(ctab_hbm.at[rc], yrows.at[i], sem)
            for k in range(NEG):
                rn = nidx[pl.ds(g * NEG + k, 16)][0]
                pltpu.async_copy(ctab_hbm.at[rn], nrows.at[i * NEG + k], sem)
            return 0

        lax.fori_loop(0, SUB, enq_body, 0)
        pltpu.make_async_copy(ctab_hbm.at[pl.ds(0, SUB), :], yrows, sem).wait()
        pltpu.make_async_copy(ctab_hbm.at[pl.ds(0, SUB * NEG), :], nrows,
                              sem).wait()

        base_c = base_w + lo
        pltpu.sync_copy(yrows, stash_out.at[pl.ds(base_c, SUB), :])
        pltpu.sync_copy(
            nrows,
            stash_out.at[pl.ds(BATCH + base_c * NEG, SUB * NEG), :])


def _sc_pos_kernel(word_hbm, wtab_hbm, stash_hbm, part_out,
                   widx, xrows, yrows, nrows, ppart, npart, sem, sems):
    wid = lax.axis_index("s") * NUM_CORES + lax.axis_index("c")
    base_w = wid * ITEMS_PER_WORKER

    pltpu.sync_copy(word_hbm.at[pl.ds(base_w, ITEMS_PER_WORKER)],
                    widx.at[pl.ds(0, ITEMS_PER_WORKER)])

    for c in range(NSUB):
        lo = c * SUB
        base_c = base_w + lo

        ycp = pltpu.async_copy(
            stash_hbm.at[pl.ds(base_c, SUB), :], yrows, sems)
        ncp = pltpu.async_copy(
            stash_hbm.at[pl.ds(BATCH + base_c * NEG, SUB * NEG), :], nrows,
            sems)

        def enq_body(i, _):
            rw = widx[pl.ds(lo + i, 16)][0]
            pltpu.async_copy(wtab_hbm.at[rw], xrows.at[i], sem)
            return 0

        lax.fori_loop(0, SUB, enq_body, 0)
        pltpu.make_async_copy(wtab_hbm.at[pl.ds(0, SUB), :], xrows, sem).wait()
        ycp.wait()
        ncp.wait()

        def item_body(i, _):
            xs = [xrows[i, pl.ds(j * 16, 16)] for j in range(NCHUNK)]
            acc = xs[0] * yrows[i, pl.ds(0, 16)]
            for j in range(1, NCHUNK):
                acc = acc + xs[j] * yrows[i, pl.ds(j * 16, 16)]
            ppart[pl.ds(i * 16, 16)] = acc
            for k in range(NEG):
                acc = xs[0] * nrows[i * NEG + k, pl.ds(0, 16)]
                for j in range(1, NCHUNK):
                    acc = acc + xs[j] * nrows[i * NEG + k, pl.ds(j * 16, 16)]

                npart[pl.ds((i * NEG + k) * 16, 16)] = acc
            return 0

        lax.fori_loop(0, SUB, item_body, 0)

        pltpu.sync_copy(ppart, part_out.at[pl.ds(base_c * 16, SUB * 16)])
        pltpu.sync_copy(
            npart,
            part_out.at[pl.ds(POS_PART + base_c * NEG * 16, SUB * NEG * 16)])


def _loss_body(part_ref, out_ref):
    x = part_ref[...]
    r = lax.broadcasted_iota(jnp.int32, (128, 8), 0) // 16
    g = lax.broadcasted_iota(jnp.int32, (128, 8), 1)
    gmat = (r == g).astype(jnp.float32)
    s = jax.lax.dot_general(x, gmat, (((1,), (0,)), ((), ())),
                            preferred_element_type=jnp.float32)
    row = lax.broadcasted_iota(jnp.int32, (PART_ROWS, 8), 0)
    z = jnp.where(row < (BATCH * 16) // 128, s, -s)
    l = jnp.minimum(z, 0.0) - jnp.log1p(jnp.exp(-jnp.abs(z)))
    out_ref[...] = jnp.full((1, 1), -jnp.sum(l) / BATCH, jnp.float32)


def kernel(word, context, negative_samples, word_embed, ctx_embed):
    negs = negative_samples.reshape(-1)

    mesh = plsc.VectorSubcoreMesh(core_axis_name="c", subcore_axis_name="s")
    params = pltpu.CompilerParams(use_tc_tiling_on_sc=True)

    kctx = functools.partial(
        pl.kernel,
        mesh=mesh,
        compiler_params=params,
        out_type=jax.ShapeDtypeStruct((STASH_ROWS, EMBED), jnp.float32),
        scratch_types=[
            pltpu.VMEM((ITEMS_PER_WORKER + 16,), jnp.int32),
            pltpu.VMEM((ITEMS_PER_WORKER * NEG + 16,), jnp.int32),
            pltpu.VMEM((SUB, EMBED), jnp.float32),
            pltpu.VMEM((SUB * NEG, EMBED), jnp.float32),
            pltpu.SemaphoreType.DMA,
        ],
    )(_sc_ctx_kernel)
    stash = kctx(context, negs, ctx_embed)

    kpos = functools.partial(
        pl.kernel,
        mesh=mesh,
        compiler_params=params,
        out_type=jax.ShapeDtypeStruct((TOTAL_PART,), jnp.float32),
        scratch_types=[
            pltpu.VMEM((ITEMS_PER_WORKER + 16,), jnp.int32),
            pltpu.VMEM((SUB, EMBED), jnp.float32),
            pltpu.VMEM((SUB, EMBED), jnp.float32),
            pltpu.VMEM((SUB * NEG, EMBED), jnp.float32),
            pltpu.VMEM((SUB * 16,), jnp.float32),
            pltpu.VMEM((SUB * NEG * 16,), jnp.float32),
            pltpu.SemaphoreType.DMA,
            pltpu.SemaphoreType.DMA,
        ],
    )(_sc_pos_kernel)
    part = kpos(word, word_embed, stash)

    loss2d = pl.pallas_call(
        _loss_body,
        out_shape=jax.ShapeDtypeStruct((1, 1), jnp.float32),
    )(part.reshape(PART_ROWS, 128))
    return loss2d[0, 0]


# speedup vs baseline: 2.2016x; 1.0380x over previous
import functools

import jax
import jax.numpy as jnp
from jax import lax
from jax.experimental import pallas as pl
from jax.experimental.pallas import tpu as pltpu
from jax.experimental.pallas import tpu_sc as plsc

VOCAB = 1000000
EMBED = 64
BATCH = 16384
NEG = 10

NUM_CORES = 2
NUM_SUBCORES = 16
NUM_WORKERS = NUM_CORES * NUM_SUBCORES
ITEMS_PER_WORKER = BATCH // NUM_WORKERS
SUB = 32
NSUB = ITEMS_PER_WORKER // SUB
NCHUNK = EMBED // 16

STASH_ROWS = BATCH * (1 + NEG)

POS_PART = BATCH * 16
TOTAL_PART = (BATCH + BATCH * NEG) * 16
PART_ROWS = TOTAL_PART // 128


def _sc_ctx_kernel(ctx_hbm, negs_hbm, ctab_hbm, stash_out,
                   cidx, nidx, yrows, nrows, sem):
    wid = lax.axis_index("s") * NUM_CORES + lax.axis_index("c")
    base_w = wid * ITEMS_PER_WORKER

    pltpu.sync_copy(ctx_hbm.at[pl.ds(base_w, ITEMS_PER_WORKER)],
                    cidx.at[pl.ds(0, ITEMS_PER_WORKER)])
    pltpu.sync_copy(negs_hbm.at[pl.ds(base_w * NEG, ITEMS_PER_WORKER * NEG)],
                    nidx.at[pl.ds(0, ITEMS_PER_WORKER * NEG)])

    for c in range(NSUB):
        lo = c * SUB

        def enq_body(i, _):
            g = lo + i
            rc = cidx[pl.ds(g, 16)][0]
            pltpu.async_copy(ctab_hbm.at[rc], yrows.at[i], sem)
            for k in range(NEG):
                rn = nidx[pl.ds(g * NEG + k, 16)][0]
                pltpu.async_copy(ctab_hbm.at[rn], nrows.at[i * NEG + k], sem)
            return 0

        lax.fori_loop(0, SUB, enq_body, 0)
        pltpu.make_async_copy(ctab_hbm.at[pl.ds(0, SUB), :], yrows, sem).wait()
        pltpu.make_async_copy(ctab_hbm.at[pl.ds(0, SUB * NEG), :], nrows,
                              sem).wait()

        base_c = base_w + lo
        pltpu.sync_copy(yrows, stash_out.at[pl.ds(base_c, SUB), :])
        pltpu.sync_copy(
            nrows,
            stash_out.at[pl.ds(BATCH + base_c * NEG, SUB * NEG), :])


def _sc_pos_kernel(word_hbm, wtab_hbm, stash_hbm, part_out,
                   widx, xrows_a, yrows_a, nrows_a, xrows_b, yrows_b, nrows_b,
                   ppart, npart, sem_a, sem_b):
    wid = lax.axis_index("s") * NUM_CORES + lax.axis_index("c")
    base_w = wid * ITEMS_PER_WORKER

    pltpu.sync_copy(word_hbm.at[pl.ds(base_w, ITEMS_PER_WORKER)],
                    widx.at[pl.ds(0, ITEMS_PER_WORKER)])

    def enqueue(c, xrows, yrows, nrows, sem):
        lo = c * SUB
        base_c = base_w + lo
        pltpu.async_copy(stash_hbm.at[pl.ds(base_c, SUB), :], yrows, sem)
        pltpu.async_copy(
            stash_hbm.at[pl.ds(BATCH + base_c * NEG, SUB * NEG), :], nrows,
            sem)

        def enq_body(i, _):
            rw = widx[pl.ds(lo + i, 16)][0]
            pltpu.async_copy(wtab_hbm.at[rw], xrows.at[i], sem)
            return 0

        lax.fori_loop(0, SUB, enq_body, 0)

    def drain(xrows, yrows, nrows, sem):
        pltpu.make_async_copy(wtab_hbm.at[pl.ds(0, SUB), :], xrows, sem).wait()
        pltpu.make_async_copy(wtab_hbm.at[pl.ds(0, SUB), :], yrows, sem).wait()
        pltpu.make_async_copy(wtab_hbm.at[pl.ds(0, SUB * NEG), :], nrows,
                              sem).wait()

    def compute(c, xrows, yrows, nrows):
        def item_body(i, _):
            xs = [xrows[i, pl.ds(j * 16, 16)] for j in range(NCHUNK)]
            acc = xs[0] * yrows[i, pl.ds(0, 16)]
            for j in range(1, NCHUNK):
                acc = acc + xs[j] * yrows[i, pl.ds(j * 16, 16)]
            ppart[pl.ds(i * 16, 16)] = acc
            for k in range(NEG):
                acc = xs[0] * nrows[i * NEG + k, pl.ds(0, 16)]
                for j in range(1, NCHUNK):
                    acc = acc + xs[j] * nrows[i * NEG + k, pl.ds(j * 16, 16)]

                npart[pl.ds((i * NEG + k) * 16, 16)] = acc
            return 0

        lax.fori_loop(0, SUB, item_body, 0)

        base_c = base_w + c * SUB
        pltpu.sync_copy(ppart, part_out.at[pl.ds(base_c * 16, SUB * 16)])
        pltpu.sync_copy(
            npart,
            part_out.at[pl.ds(POS_PART + base_c * NEG * 16, SUB * NEG * 16)])

    enqueue(0, xrows_a, yrows_a, nrows_a, sem_a)

    def pair_body(tt, _):
        c = tt * 2
        enqueue(c + 1, xrows_b, yrows_b, nrows_b, sem_b)
        drain(xrows_a, yrows_a, nrows_a, sem_a)
        compute(c, xrows_a, yrows_a, nrows_a)

        @pl.when(tt < NSUB // 2 - 1)
        def _():
            enqueue(c + 2, xrows_a, yrows_a, nrows_a, sem_a)
        drain(xrows_b, yrows_b, nrows_b, sem_b)
        compute(c + 1, xrows_b, yrows_b, nrows_b)
        return 0

    lax.fori_loop(0, NSUB // 2, pair_body, 0)


def _loss_body(part_ref, out_ref):
    x = part_ref[...]
    r = lax.broadcasted_iota(jnp.int32, (128, 8), 0) // 16
    g = lax.broadcasted_iota(jnp.int32, (128, 8), 1)
    gmat = (r == g).astype(jnp.float32)
    s = jax.lax.dot_general(x, gmat, (((1,), (0,)), ((), ())),
                            preferred_element_type=jnp.float32)
    row = lax.broadcasted_iota(jnp.int32, (PART_ROWS, 8), 0)
    z = jnp.where(row < (BATCH * 16) // 128, s, -s)
    l = jnp.minimum(z, 0.0) - jnp.log1p(jnp.exp(-jnp.abs(z)))
    out_ref[...] = jnp.full((1, 1), -jnp.sum(l) / BATCH, jnp.float32)


def kernel(word, context, negative_samples, word_embed, ctx_embed):
    negs = negative_samples.reshape(-1)

    mesh = plsc.VectorSubcoreMesh(core_axis_name="c", subcore_axis_name="s")
    params = pltpu.CompilerParams(use_tc_tiling_on_sc=True)

    kctx = functools.partial(
        pl.kernel,
        mesh=mesh,
        compiler_params=params,
        out_type=jax.ShapeDtypeStruct((STASH_ROWS, EMBED), jnp.float32),
        scratch_types=[
            pltpu.VMEM((ITEMS_PER_WORKER + 16,), jnp.int32),
            pltpu.VMEM((ITEMS_PER_WORKER * NEG + 16,), jnp.int32),
            pltpu.VMEM((SUB, EMBED), jnp.float32),
            pltpu.VMEM((SUB * NEG, EMBED), jnp.float32),
            pltpu.SemaphoreType.DMA,
        ],
    )(_sc_ctx_kernel)
    stash = kctx(context, negs, ctx_embed)

    kpos = functools.partial(
        pl.kernel,
        mesh=mesh,
        compiler_params=params,
        out_type=jax.ShapeDtypeStruct((TOTAL_PART,), jnp.float32),
        scratch_types=[
            pltpu.VMEM((ITEMS_PER_WORKER + 16,), jnp.int32),
            pltpu.VMEM((SUB, EMBED), jnp.float32),
            pltpu.VMEM((SUB, EMBED), jnp.float32),
            pltpu.VMEM((SUB * NEG, EMBED), jnp.float32),
            pltpu.VMEM((SUB, EMBED), jnp.float32),
            pltpu.VMEM((SUB, EMBED), jnp.float32),
            pltpu.VMEM((SUB * NEG, EMBED), jnp.float32),
            pltpu.VMEM((SUB * 16,), jnp.float32),
            pltpu.VMEM((SUB * NEG * 16,), jnp.float32),
            pltpu.SemaphoreType.DMA,
            pltpu.SemaphoreType.DMA,
        ],
    )(_sc_pos_kernel)
    part = kpos(word, word_embed, stash)

    loss2d = pl.pallas_call(
        _loss_body,
        out_shape=jax.ShapeDtypeStruct((1, 1), jnp.float32),
    )(part.reshape(PART_ROWS, 128))
    return loss2d[0, 0]
